# baseline (speedup 1.0000x reference)
import functools

import jax
import jax.numpy as jnp
from jax import lax
from jax.experimental import pallas as pl
from jax.experimental.pallas import tpu as pltpu
from jax.experimental.pallas import tpu_sc as plsc

D_MODEL = 64
NUM_DAYS = 7
BATCH = 16384


@functools.cache
def _build_gather():
    info = plsc.get_sparse_core_info()
    num_cores, num_subcores = info.num_cores, info.num_subcores
    num_workers = num_cores * num_subcores
    b_per_w = BATCH // num_workers
    mesh = plsc.VectorSubcoreMesh(core_axis_name="c", subcore_axis_name="s")

    @functools.partial(
        pl.kernel,
        mesh=mesh,
        out_type=jax.ShapeDtypeStruct((BATCH, D_MODEL), jnp.float32),
        scratch_types=[
            pltpu.VMEM((b_per_w,), jnp.int32),
            pltpu.VMEM((b_per_w, D_MODEL), jnp.float32),
            pltpu.SemaphoreType.DMA,
        ],
        compiler_params=pltpu.CompilerParams(use_tc_tiling_on_sc=False),
    )
    def gather_kernel(idx_hbm, table_hbm, out_hbm, idx_v, rows_v, sem):
        wid = lax.axis_index("s") * num_cores + lax.axis_index("c")
        base = wid * b_per_w
        pltpu.sync_copy(idx_hbm.at[pl.ds(base, b_per_w)], idx_v)
        pltpu.async_copy(table_hbm.at[idx_v], rows_v, sem).wait()
        pltpu.sync_copy(rows_v, out_hbm.at[pl.ds(base, b_per_w)])

    return gather_kernel


def kernel(day_indices, table):
    return _build_gather()(day_indices.astype(jnp.int32), table)


# speedup vs baseline: 2.8574x; 2.8574x over previous
import functools

import jax
import jax.numpy as jnp
from jax import lax
from jax.experimental import pallas as pl
from jax.experimental.pallas import tpu as pltpu
from jax.experimental.pallas import tpu_sc as plsc

D_MODEL = 64
NUM_DAYS = 7
BATCH = 16384


@functools.cache
def _build_gather():
    info = plsc.get_sparse_core_info()
    num_cores, num_subcores = info.num_cores, info.num_subcores
    num_workers = num_cores * num_subcores
    b_per_w = BATCH // num_workers
    mesh = plsc.VectorSubcoreMesh(core_axis_name="c", subcore_axis_name="s")

    @functools.partial(
        pl.kernel,
        mesh=mesh,
        out_type=jax.ShapeDtypeStruct((BATCH, D_MODEL), jnp.float32),
        scratch_types=[
            pltpu.VMEM((b_per_w,), jnp.int32),
            pltpu.VMEM((b_per_w, D_MODEL), jnp.float32),
            pltpu.SemaphoreType.DMA,
        ],
        compiler_params=pltpu.CompilerParams(use_tc_tiling_on_sc=False),
    )
    def gather_kernel(idx_hbm, table_hbm, out_hbm, idx_v, rows_v, sem):
        wid = lax.axis_index("s") * num_cores + lax.axis_index("c")
        base = wid * b_per_w
        pltpu.sync_copy(idx_hbm.at[pl.ds(base, b_per_w)], idx_v)
        off = wid * NUM_DAYS
        for k in range(b_per_w // 16):
            sl = pl.ds(k * 16, 16)
            idx_v[sl] = idx_v[sl] + off
        pltpu.async_copy(table_hbm.at[idx_v], rows_v, sem).wait()
        pltpu.sync_copy(rows_v, out_hbm.at[pl.ds(base, b_per_w)])

    return gather_kernel


def kernel(day_indices, table):
    info = plsc.get_sparse_core_info()
    table_rep = jnp.tile(table, (info.num_cores * info.num_subcores, 1))
    return _build_gather()(day_indices.astype(jnp.int32), table_rep)


# speedup vs baseline: 2.8817x; 1.0085x over previous
import functools

import jax
import jax.numpy as jnp
from jax import lax
from jax.experimental import pallas as pl
from jax.experimental.pallas import tpu as pltpu
from jax.experimental.pallas import tpu_sc as plsc

D_MODEL = 64
NUM_DAYS = 7
BATCH = 16384


@functools.cache
def _build_gather():
    info = plsc.get_sparse_core_info()
    num_cores, num_subcores = info.num_cores, info.num_subcores
    num_workers = num_cores * num_subcores
    b_per_w = BATCH // num_workers
    mesh = plsc.VectorSubcoreMesh(core_axis_name="c", subcore_axis_name="s")

    @functools.partial(
        pl.kernel,
        mesh=mesh,
        out_type=jax.ShapeDtypeStruct((BATCH, D_MODEL), jnp.float32),
        scratch_types=[
            pltpu.VMEM((b_per_w,), jnp.int32),
            pltpu.VMEM((b_per_w, D_MODEL), jnp.float32),
            pltpu.SemaphoreType.DMA,
        ],
        compiler_params=pltpu.CompilerParams(use_tc_tiling_on_sc=False),
    )
    def gather_kernel(idx_hbm, table_hbm, out_hbm, idx_v, rows_v, sem):
        wid = lax.axis_index("s") * num_cores + lax.axis_index("c")
        base = wid * b_per_w
        pltpu.sync_copy(idx_hbm.at[pl.ds(base, b_per_w)], idx_v)
        replica = table_hbm.at[pl.ds(wid * NUM_DAYS, NUM_DAYS)]
        pltpu.async_copy(replica.at[idx_v], rows_v, sem).wait()
        pltpu.sync_copy(rows_v, out_hbm.at[pl.ds(base, b_per_w)])

    return gather_kernel


def kernel(day_indices, table):
    info = plsc.get_sparse_core_info()
    table_rep = jnp.tile(table, (info.num_cores * info.num_subcores, 1))
    return _build_gather()(day_indices.astype(jnp.int32), table_rep)


# speedup vs baseline: 2.9639x; 1.0285x over previous
import functools

import jax
import jax.numpy as jnp
from jax import lax
from jax.experimental import pallas as pl
from jax.experimental.pallas import tpu as pltpu
from jax.experimental.pallas import tpu_sc as plsc

D_MODEL = 64
NUM_DAYS = 7
NUM_PAIRS = NUM_DAYS * NUM_DAYS
PAIR_ROWS = 56
BATCH = 16384


@functools.cache
def _build_gather():
    info = plsc.get_sparse_core_info()
    num_cores, num_subcores = info.num_cores, info.num_subcores
    num_workers = num_cores * num_subcores
    b_per_w = BATCH // num_workers
    p_per_w = b_per_w // 2
    mesh = plsc.VectorSubcoreMesh(core_axis_name="c", subcore_axis_name="s")

    @functools.partial(
        pl.kernel,
        mesh=mesh,
        out_type=jax.ShapeDtypeStruct((BATCH // 2, 2 * D_MODEL), jnp.float32),
        scratch_types=[
            pltpu.VMEM((p_per_w,), jnp.int32),
            pltpu.VMEM((p_per_w,), jnp.int32),
            pltpu.VMEM((p_per_w,), jnp.int32),
            pltpu.VMEM((p_per_w, 2 * D_MODEL), jnp.float32),
            pltpu.SemaphoreType.DMA,
        ],
    )
    def gather_kernel(ev_hbm, od_hbm, table2_hbm, out_hbm, ev_v, od_v, pidx_v, rows_v, sem):
        wid = lax.axis_index("s") * num_cores + lax.axis_index("c")
        base = wid * p_per_w
        pltpu.sync_copy(ev_hbm.at[pl.ds(base, p_per_w)], ev_v)
        pltpu.sync_copy(od_hbm.at[pl.ds(base, p_per_w)], od_v)
        for k in range(p_per_w // 16):
            sl = pl.ds(k * 16, 16)
            pidx_v[sl] = ev_v[sl] * NUM_DAYS + od_v[sl]
        replica = table2_hbm.at[pl.ds(wid * PAIR_ROWS, PAIR_ROWS)]
        pltpu.async_copy(replica.at[pidx_v], rows_v, sem).wait()
        pltpu.sync_copy(rows_v, out_hbm.at[pl.ds(wid * p_per_w, p_per_w)])

    return gather_kernel


def kernel(day_indices, table):
    info = plsc.get_sparse_core_info()
    table2 = jnp.concatenate(
        [jnp.repeat(table, NUM_DAYS, axis=0), jnp.tile(table, (NUM_DAYS, 1))],
        axis=1,
    )
    table2 = jnp.pad(table2, ((0, PAIR_ROWS - NUM_PAIRS), (0, 0)))
    table2_rep = jnp.tile(table2, (info.num_cores * info.num_subcores, 1))
    idx = day_indices.astype(jnp.int32)
    out2 = _build_gather()(idx[0::2], idx[1::2], table2_rep)
    return out2.reshape(BATCH, D_MODEL)


# speedup vs baseline: 3.4661x; 1.1695x over previous
import functools

import jax
import jax.numpy as jnp
from jax import lax
from jax.experimental import pallas as pl
from jax.experimental.pallas import tpu as pltpu
from jax.experimental.pallas import tpu_sc as plsc

D_MODEL = 64
NUM_DAYS = 7
NUM_PAIRS = NUM_DAYS * NUM_DAYS
PAIR_ROWS = 56
BATCH = 16384
HALF = BATCH // 2


@functools.cache
def _build_gather():
    info = plsc.get_sparse_core_info()
    num_cores, num_subcores = info.num_cores, info.num_subcores
    num_workers = num_cores * num_subcores
    p_per_w = HALF // num_workers
    mesh = plsc.VectorSubcoreMesh(core_axis_name="c", subcore_axis_name="s")

    @functools.partial(
        pl.kernel,
        mesh=mesh,
        out_type=jax.ShapeDtypeStruct((HALF, 2 * D_MODEL), jnp.float32),
        scratch_types=[
            pltpu.VMEM((p_per_w,), jnp.int32),
            pltpu.VMEM((p_per_w,), jnp.int32),
            pltpu.VMEM((p_per_w,), jnp.int32),
            pltpu.VMEM((p_per_w, 2 * D_MODEL), jnp.float32),
            pltpu.SemaphoreType.DMA,
        ],
    )
    def gather_kernel(idx_hbm, table2_hbm, out_hbm, ev_v, od_v, pidx_v, rows_v, sem):
        wid = lax.axis_index("s") * num_cores + lax.axis_index("c")
        base = wid * p_per_w
        pltpu.sync_copy(idx_hbm.at[pl.ds(base, p_per_w)], ev_v)
        pltpu.sync_copy(idx_hbm.at[pl.ds(HALF + base, p_per_w)], od_v)
        for k in range(p_per_w // 16):
            sl = pl.ds(k * 16, 16)
            pidx_v[sl] = ev_v[sl] * NUM_DAYS + od_v[sl]
        replica = table2_hbm.at[pl.ds(wid * PAIR_ROWS, PAIR_ROWS)]
        pltpu.async_copy(replica.at[pidx_v], rows_v, sem).wait()
        pltpu.sync_copy(rows_v, out_hbm.at[pl.ds(base, p_per_w)])

    return gather_kernel


def _transpose_body(x_ref, y_ref, t_ref):
    h = pl.program_id(1)

    @pl.when(h == 0)
    def _():
        t_ref[...] = jnp.swapaxes(x_ref[...], 0, 1)

    y_ref[...] = t_ref[pl.ds(h * D_MODEL, D_MODEL), :]


@functools.cache
def _build_transpose():
    blk = 1024
    n_blk = HALF // blk
    return pl.pallas_call(
        _transpose_body,
        grid=(n_blk, 2),
        in_specs=[pl.BlockSpec((blk, 2 * D_MODEL), lambda k, h: (k, 0))],
        out_specs=pl.BlockSpec((D_MODEL, blk), lambda k, h: (0, h * n_blk + k)),
        out_shape=jax.ShapeDtypeStruct((D_MODEL, BATCH), jnp.float32),
        scratch_shapes=[pltpu.VMEM((2 * D_MODEL, blk), jnp.float32)],
    )


def kernel(day_indices, table):
    info = plsc.get_sparse_core_info()
    table2 = jnp.concatenate(
        [jnp.repeat(table, NUM_DAYS, axis=0), jnp.tile(table, (NUM_DAYS, 1))],
        axis=1,
    )
    table2 = jnp.pad(table2, ((0, PAIR_ROWS - NUM_PAIRS), (0, 0)))
    table2_rep = jnp.tile(table2, (info.num_cores * info.num_subcores, 1))
    pairs = _build_gather()(day_indices.astype(jnp.int32), table2_rep)
    y = _build_transpose()(pairs)
    return jnp.transpose(y)


# speedup vs baseline: 3.9386x; 1.1363x over previous
import functools

import jax
import jax.numpy as jnp
from jax import lax
from jax.experimental import pallas as pl
from jax.experimental.pallas import tpu as pltpu
from jax.experimental.pallas import tpu_sc as plsc

D_MODEL = 64
NUM_DAYS = 7
NUM_PAIRS = NUM_DAYS * NUM_DAYS
PAIR_ROWS = 56
BATCH = 16384
HALF = BATCH // 2
BLK = 1024


@functools.cache
def _build_gather():
    info = plsc.get_sparse_core_info()
    num_cores, num_subcores = info.num_cores, info.num_subcores
    num_workers = num_cores * num_subcores
    p_per_w = HALF // num_workers
    w_per_blk = BLK // p_per_w
    mesh = plsc.VectorSubcoreMesh(core_axis_name="c", subcore_axis_name="s")

    @functools.partial(
        pl.kernel,
        mesh=mesh,
        out_type=jax.ShapeDtypeStruct((HALF, 2 * D_MODEL), jnp.float32),
        scratch_types=[
            pltpu.VMEM((p_per_w,), jnp.int32),
            pltpu.VMEM((p_per_w,), jnp.int32),
            pltpu.VMEM((p_per_w,), jnp.int32),
            pltpu.VMEM((p_per_w, 2 * D_MODEL), jnp.float32),
            pltpu.SemaphoreType.DMA,
        ],
    )
    def gather_kernel(idx_hbm, table2_hbm, out_hbm, ev_v, od_v, pidx_v, rows_v, sem):
        wid = lax.axis_index("s") * num_cores + lax.axis_index("c")
        e_base = (wid // w_per_blk) * (2 * BLK) + (wid % w_per_blk) * p_per_w
        pltpu.sync_copy(idx_hbm.at[pl.ds(e_base, p_per_w)], ev_v)
        pltpu.sync_copy(idx_hbm.at[pl.ds(e_base + BLK, p_per_w)], od_v)
        for k in range(p_per_w // 16):
            sl = pl.ds(k * 16, 16)
            pidx_v[sl] = ev_v[sl] * NUM_DAYS + od_v[sl]
        replica = table2_hbm.at[pl.ds(wid * PAIR_ROWS, PAIR_ROWS)]
        pltpu.async_copy(replica.at[pidx_v], rows_v, sem).wait()
        pltpu.sync_copy(rows_v, out_hbm.at[pl.ds(wid * p_per_w, p_per_w)])

    return gather_kernel


def _transpose_body(x_ref, y_ref):
    t = jnp.swapaxes(x_ref[...], 0, 1)
    y_ref[:, :BLK] = t[:D_MODEL]
    y_ref[:, BLK:] = t[D_MODEL:]


@functools.cache
def _build_transpose():
    n_blk = HALF // BLK
    return pl.pallas_call(
        _transpose_body,
        grid=(n_blk,),
        in_specs=[pl.BlockSpec((BLK, 2 * D_MODEL), lambda k: (k, 0))],
        out_specs=pl.BlockSpec((D_MODEL, 2 * BLK), lambda k: (0, k)),
        out_shape=jax.ShapeDtypeStruct((D_MODEL, BATCH), jnp.float32),
    )


def kernel(day_indices, table):
    info = plsc.get_sparse_core_info()
    table2 = jnp.concatenate(
        [jnp.repeat(table, NUM_DAYS, axis=0), jnp.tile(table, (NUM_DAYS, 1))],
        axis=1,
    )
    table2 = jnp.pad(table2, ((0, PAIR_ROWS - NUM_PAIRS), (0, 0)))
    table2_rep = jnp.tile(table2, (info.num_cores * info.num_subcores, 1))
    pairs = _build_gather()(day_indices.astype(jnp.int32), table2_rep)
    y = _build_transpose()(pairs)
    return jnp.transpose(y)


# speedup vs baseline: 4.2729x; 1.0849x over previous
import functools

import jax
import jax.numpy as jnp
from jax import lax
from jax.experimental import pallas as pl
from jax.experimental.pallas import tpu as pltpu
from jax.experimental.pallas import tpu_sc as plsc

D_MODEL = 64
NUM_DAYS = 7
NUM_PAIRS = NUM_DAYS * NUM_DAYS
PAIR_ROWS = 56
BATCH = 16384
HALF = BATCH // 2
BLK = 2048


@functools.cache
def _build_gather():
    info = plsc.get_sparse_core_info()
    num_cores, num_subcores = info.num_cores, info.num_subcores
    num_workers = num_cores * num_subcores
    p_per_w = HALF // num_workers
    w_per_blk = BLK // p_per_w
    mesh = plsc.VectorSubcoreMesh(core_axis_name="c", subcore_axis_name="s")

    @functools.partial(
        pl.kernel,
        mesh=mesh,
        out_type=jax.ShapeDtypeStruct((HALF, 2 * D_MODEL), jnp.float32),
        scratch_types=[
            pltpu.VMEM((p_per_w,), jnp.int32),
            pltpu.VMEM((p_per_w,), jnp.int32),
            pltpu.VMEM((p_per_w,), jnp.int32),
            pltpu.VMEM((p_per_w, 2 * D_MODEL), jnp.float32),
            pltpu.SemaphoreType.DMA,
        ],
    )
    def gather_kernel(idx_hbm, table2_hbm, out_hbm, ev_v, od_v, pidx_v, rows_v, sem):
        wid = lax.axis_index("s") * num_cores + lax.axis_index("c")
        e_base = (wid // w_per_blk) * (2 * BLK) + (wid % w_per_blk) * p_per_w
        pltpu.sync_copy(idx_hbm.at[pl.ds(e_base, p_per_w)], ev_v)
        pltpu.sync_copy(idx_hbm.at[pl.ds(e_base + BLK, p_per_w)], od_v)
        for k in range(p_per_w // 16):
            sl = pl.ds(k * 16, 16)
            pidx_v[sl] = ev_v[sl] * NUM_DAYS + od_v[sl]
        replica = table2_hbm.at[pl.ds(wid * PAIR_ROWS, PAIR_ROWS)]
        pltpu.async_copy(replica.at[pidx_v], rows_v, sem).wait()
        pltpu.sync_copy(rows_v, out_hbm.at[pl.ds(wid * p_per_w, p_per_w)])

    return gather_kernel


def _transpose_body(x_ref, y_ref):
    t = jnp.swapaxes(x_ref[...], 0, 1)
    y_ref[:, :BLK] = t[:D_MODEL]
    y_ref[:, BLK:] = t[D_MODEL:]


@functools.cache
def _build_transpose():
    n_blk = HALF // BLK
    return pl.pallas_call(
        _transpose_body,
        grid=(n_blk,),
        in_specs=[pl.BlockSpec((BLK, 2 * D_MODEL), lambda k: (k, 0))],
        out_specs=pl.BlockSpec((D_MODEL, 2 * BLK), lambda k: (0, k)),
        out_shape=jax.ShapeDtypeStruct((D_MODEL, BATCH), jnp.float32),
    )


def kernel(day_indices, table):
    info = plsc.get_sparse_core_info()
    table2 = jnp.concatenate(
        [jnp.repeat(table, NUM_DAYS, axis=0), jnp.tile(table, (NUM_DAYS, 1))],
        axis=1,
    )
    table2 = jnp.pad(table2, ((0, PAIR_ROWS - NUM_PAIRS), (0, 0)))
    table2_rep = jnp.tile(table2, (info.num_cores * info.num_subcores, 1))
    pairs = _build_gather()(day_indices.astype(jnp.int32), table2_rep)
    y = _build_transpose()(pairs)
    return jnp.transpose(y)


# speedup vs baseline: 4.4068x; 1.0313x over previous
import functools

import jax
import jax.numpy as jnp
from jax import lax
from jax.experimental import pallas as pl
from jax.experimental.pallas import tpu as pltpu
from jax.experimental.pallas import tpu_sc as plsc

D_MODEL = 64
NUM_DAYS = 7
NUM_PAIRS = NUM_DAYS * NUM_DAYS
PAIR_ROWS = 56
BATCH = 16384
HALF = BATCH // 2
BLK = 4096


@functools.cache
def _build_gather():
    info = plsc.get_sparse_core_info()
    num_cores, num_subcores = info.num_cores, info.num_subcores
    num_workers = num_cores * num_subcores
    p_per_w = HALF // num_workers
    w_per_blk = BLK // p_per_w
    mesh = plsc.VectorSubcoreMesh(core_axis_name="c", subcore_axis_name="s")

    @functools.partial(
        pl.kernel,
        mesh=mesh,
        out_type=jax.ShapeDtypeStruct((HALF, 2 * D_MODEL), jnp.float32),
        scratch_types=[
            pltpu.VMEM((p_per_w,), jnp.int32),
            pltpu.VMEM((p_per_w,), jnp.int32),
            pltpu.VMEM((p_per_w,), jnp.int32),
            pltpu.VMEM((p_per_w, 2 * D_MODEL), jnp.float32),
            pltpu.SemaphoreType.DMA,
        ],
    )
    def gather_kernel(idx_hbm, table2_hbm, out_hbm, ev_v, od_v, pidx_v, rows_v, sem):
        wid = lax.axis_index("s") * num_cores + lax.axis_index("c")
        e_base = (wid // w_per_blk) * (2 * BLK) + (wid % w_per_blk) * p_per_w
        pltpu.sync_copy(idx_hbm.at[pl.ds(e_base, p_per_w)], ev_v)
        pltpu.sync_copy(idx_hbm.at[pl.ds(e_base + BLK, p_per_w)], od_v)
        for k in range(p_per_w // 16):
            sl = pl.ds(k * 16, 16)
            pidx_v[sl] = ev_v[sl] * NUM_DAYS + od_v[sl]
        replica = table2_hbm.at[pl.ds(wid * PAIR_ROWS, PAIR_ROWS)]
        pltpu.async_copy(replica.at[pidx_v], rows_v, sem).wait()
        pltpu.sync_copy(rows_v, out_hbm.at[pl.ds(wid * p_per_w, p_per_w)])

    return gather_kernel


def _transpose_body(x_ref, y_ref):
    t = jnp.swapaxes(x_ref[...], 0, 1)
    y_ref[:, :BLK] = t[:D_MODEL]
    y_ref[:, BLK:] = t[D_MODEL:]


@functools.cache
def _build_transpose():
    n_blk = HALF // BLK
    return pl.pallas_call(
        _transpose_body,
        grid=(n_blk,),
        in_specs=[pl.BlockSpec((BLK, 2 * D_MODEL), lambda k: (k, 0))],
        out_specs=pl.BlockSpec((D_MODEL, 2 * BLK), lambda k: (0, k)),
        out_shape=jax.ShapeDtypeStruct((D_MODEL, BATCH), jnp.float32),
    )


def kernel(day_indices, table):
    info = plsc.get_sparse_core_info()
    table2 = jnp.concatenate(
        [jnp.repeat(table, NUM_DAYS, axis=0), jnp.tile(table, (NUM_DAYS, 1))],
        axis=1,
    )
    table2 = jnp.pad(table2, ((0, PAIR_ROWS - NUM_PAIRS), (0, 0)))
    table2_rep = jnp.tile(table2, (info.num_cores * info.num_subcores, 1))
    pairs = _build_gather()(day_indices.astype(jnp.int32), table2_rep)
    y = _build_transpose()(pairs)
    return jnp.transpose(y)


# speedup vs baseline: 4.6455x; 1.0542x over previous
import functools

import jax
import jax.numpy as jnp
from jax import lax
from jax.experimental import pallas as pl
from jax.experimental.pallas import tpu as pltpu
from jax.experimental.pallas import tpu_sc as plsc

D_MODEL = 64
NUM_DAYS = 7
NUM_PAIRS = NUM_DAYS * NUM_DAYS
PAIR_ROWS = 56
BATCH = 16384
HALF = BATCH // 2
BLK = 4096


@functools.cache
def _build_gather():
    info = plsc.get_sparse_core_info()
    num_cores, num_subcores = info.num_cores, info.num_subcores
    num_workers = num_cores * num_subcores
    p_per_w = HALF // num_workers
    w_per_blk = BLK // p_per_w
    mesh = plsc.VectorSubcoreMesh(core_axis_name="c", subcore_axis_name="s")

    @functools.partial(
        pl.kernel,
        mesh=mesh,
        out_type=jax.ShapeDtypeStruct((HALF, 2 * D_MODEL), jnp.float32),
        scratch_types=[
            pltpu.VMEM((p_per_w,), jnp.int32),
            pltpu.VMEM((p_per_w,), jnp.int32),
            pltpu.VMEM((p_per_w,), jnp.int32),
            pltpu.VMEM((p_per_w, 2 * D_MODEL), jnp.float32),
            pltpu.VMEM_SHARED((PAIR_ROWS, 2 * D_MODEL), jnp.float32),
            pltpu.SemaphoreType.DMA,
        ],
    )
    def gather_kernel(idx_hbm, table2_hbm, out_hbm, ev_v, od_v, pidx_v, rows_v,
                      table2_spm, sem):
        wid = lax.axis_index("s") * num_cores + lax.axis_index("c")
        e_base = (wid // w_per_blk) * (2 * BLK) + (wid % w_per_blk) * p_per_w
        pltpu.sync_copy(idx_hbm.at[pl.ds(e_base, p_per_w)], ev_v)
        pltpu.sync_copy(idx_hbm.at[pl.ds(e_base + BLK, p_per_w)], od_v)

        @pl.when(lax.axis_index("s") == 0)
        def _():
            pltpu.sync_copy(table2_hbm, table2_spm)

        for k in range(p_per_w // 16):
            sl = pl.ds(k * 16, 16)
            pidx_v[sl] = ev_v[sl] * NUM_DAYS + od_v[sl]
        plsc.subcore_barrier()
        pltpu.async_copy(table2_spm.at[pidx_v], rows_v, sem).wait()
        pltpu.sync_copy(rows_v, out_hbm.at[pl.ds(wid * p_per_w, p_per_w)])

    return gather_kernel


def _transpose_body(x_ref, y_ref):
    t = jnp.swapaxes(x_ref[...], 0, 1)
    y_ref[:, :BLK] = t[:D_MODEL]
    y_ref[:, BLK:] = t[D_MODEL:]


@functools.cache
def _build_transpose():
    n_blk = HALF // BLK
    return pl.pallas_call(
        _transpose_body,
        grid=(n_blk,),
        in_specs=[pl.BlockSpec((BLK, 2 * D_MODEL), lambda k: (k, 0))],
        out_specs=pl.BlockSpec((D_MODEL, 2 * BLK), lambda k: (0, k)),
        out_shape=jax.ShapeDtypeStruct((D_MODEL, BATCH), jnp.float32),
    )


def kernel(day_indices, table):
    table2 = jnp.concatenate(
        [jnp.repeat(table, NUM_DAYS, axis=0), jnp.tile(table, (NUM_DAYS, 1))],
        axis=1,
    )
    table2 = jnp.pad(table2, ((0, PAIR_ROWS - NUM_PAIRS), (0, 0)))
    pairs = _build_gather()(day_indices.astype(jnp.int32), table2)
    y = _build_transpose()(pairs)
    return jnp.transpose(y)
